# baseline (speedup 1.0000x reference)
import functools

import jax
import jax.numpy as jnp
from jax import lax
from jax.experimental import pallas as pl
from jax.experimental.pallas import tpu as pltpu
from jax.experimental.pallas import tpu_sc as plsc

BATCH = 16384
NFACT = 128
H1 = 512
H2 = 256
NCHUNK = 2
CHUNK = BATCH // NCHUNK


def _make_sc_gather(chunk_rows, chunk_off, D):
    info = plsc.get_sparse_core_info()
    NC, NS = info.num_cores, info.num_subcores
    NW = NC * NS
    assert chunk_rows % (8 * NW) == 0
    b_per_w = chunk_rows // NW
    mesh = plsc.VectorSubcoreMesh(core_axis_name="c", subcore_axis_name="s")

    half = b_per_w // 2

    @functools.partial(
        pl.kernel,
        mesh=mesh,
        out_type=[
            jax.ShapeDtypeStruct((chunk_rows, D), jnp.float32),
            jax.ShapeDtypeStruct((chunk_rows, D), jnp.float32),
        ],
        scratch_types=[
            pltpu.VMEM((half,), jnp.int32),
            pltpu.VMEM((half,), jnp.int32),
            pltpu.VMEM((half,), jnp.int32),
            pltpu.VMEM((half,), jnp.int32),
            pltpu.VMEM((half, D), jnp.float32),
            pltpu.VMEM((half, D), jnp.float32),
            pltpu.VMEM((half, D), jnp.float32),
            pltpu.VMEM((half, D), jnp.float32),
            pltpu.SemaphoreType.DMA,
            pltpu.SemaphoreType.DMA,
            pltpu.SemaphoreType.DMA,
            pltpu.SemaphoreType.DMA,
            pltpu.SemaphoreType.DMA,
        ],
    )
    def gather_k(user_hbm, item_hbm, ut_hbm, it_hbm, ue_out, ie_out,
                 ui0, ui1, ii0, ii1, ur0, ur1, ir0, ir1,
                 gs0, gs1, gs2, gs3, ssem):
        wid = lax.axis_index("s") * NC + lax.axis_index("c")
        src = chunk_off + wid * b_per_w
        dst = wid * b_per_w
        pltpu.sync_copy(user_hbm.at[pl.ds(src, half)], ui0)
        pltpu.sync_copy(item_hbm.at[pl.ds(src, half)], ii0)
        pltpu.sync_copy(user_hbm.at[pl.ds(src + half, half)], ui1)
        pltpu.sync_copy(item_hbm.at[pl.ds(src + half, half)], ii1)
        g0 = pltpu.async_copy(ut_hbm.at[ui0], ur0, gs0)
        g1 = pltpu.async_copy(it_hbm.at[ii0], ir0, gs1)
        g2 = pltpu.async_copy(ut_hbm.at[ui1], ur1, gs2)
        g3 = pltpu.async_copy(it_hbm.at[ii1], ir1, gs3)
        g0.wait()
        s0 = pltpu.async_copy(ur0, ue_out.at[pl.ds(dst, half)], ssem)
        g1.wait()
        s1 = pltpu.async_copy(ir0, ie_out.at[pl.ds(dst, half)], ssem)
        g2.wait()
        s2 = pltpu.async_copy(ur1, ue_out.at[pl.ds(dst + half, half)], ssem)
        g3.wait()
        s3 = pltpu.async_copy(ir1, ie_out.at[pl.ds(dst + half, half)], ssem)
        s0.wait()
        s1.wait()
        s2.wait()
        s3.wait()

    return gather_k


_sc_gathers = [_make_sc_gather(CHUNK, c * CHUNK, NFACT) for c in range(NCHUNK)]


def _mlp_body(ue, ie, w1a, w1b, b1, w2, b2, w3r, b3, out):
    x = jnp.dot(ue[...], w1a[...], preferred_element_type=jnp.float32)
    x = x + jnp.dot(ie[...], w1b[...], preferred_element_type=jnp.float32)
    h1 = jnp.maximum(x + b1[...], 0.0)
    h2 = jnp.dot(h1, w2[...], preferred_element_type=jnp.float32) + b2[...]
    h2 = jnp.maximum(h2, 0.0)
    o = jax.lax.dot_general(w3r[...], h2, (((1,), (1,)), ((), ())),
                            preferred_element_type=jnp.float32)
    out[...] = o.reshape(out.shape) + b3[0, 0]


def _mlp(ue, ie, W1, b1r, W2, b2r, w3r, b3r, block_m=2048):
    B = ue.shape[0]
    grid = (B // block_m,)
    return pl.pallas_call(
        _mlp_body,
        grid=grid,
        in_specs=[
            pl.BlockSpec((block_m, NFACT), lambda i: (i, 0)),
            pl.BlockSpec((block_m, NFACT), lambda i: (i, 0)),
            pl.BlockSpec((NFACT, H1), lambda i: (0, 0)),
            pl.BlockSpec((NFACT, H1), lambda i: (1, 0)),
            pl.BlockSpec((1, H1), lambda i: (0, 0)),
            pl.BlockSpec((H1, H2), lambda i: (0, 0)),
            pl.BlockSpec((1, H2), lambda i: (0, 0)),
            pl.BlockSpec((1, H2), lambda i: (0, 0)),
            pl.BlockSpec((1, 1), lambda i: (0, 0)),
        ],
        out_specs=pl.BlockSpec((block_m,), lambda i: (i,)),
        out_shape=jax.ShapeDtypeStruct((B,), jnp.float32),
    )(ue, ie, W1, W1, b1r, W2, b2r, w3r, b3r)


@jax.jit
def kernel(user, item, user_table, item_table, W1, b1, W2, b2, W3, b3):
    b1r = b1.reshape(1, H1)
    b2r = b2.reshape(1, H2)
    w3r = W3.reshape(1, H2)
    b3r = b3.reshape(1, 1)
    embs = [g(user, item, user_table, item_table) for g in _sc_gathers]
    outs = [_mlp(ue, ie, W1, b1r, W2, b2r, w3r, b3r) for ue, ie in embs]
    return jnp.concatenate(outs)


# speedup vs baseline: 1.0117x; 1.0117x over previous
import functools

import jax
import jax.numpy as jnp
from jax import lax
from jax.experimental import pallas as pl
from jax.experimental.pallas import tpu as pltpu
from jax.experimental.pallas import tpu_sc as plsc

BATCH = 16384
NFACT = 128
H1 = 512
H2 = 256
CHUNK_SIZES = (6144, 10240)


def _make_sc_gather(chunk_rows, chunk_off, D):
    info = plsc.get_sparse_core_info()
    NC, NS = info.num_cores, info.num_subcores
    NW = NC * NS
    assert chunk_rows % (8 * NW) == 0
    b_per_w = chunk_rows // NW
    mesh = plsc.VectorSubcoreMesh(core_axis_name="c", subcore_axis_name="s")

    @functools.partial(
        pl.kernel,
        mesh=mesh,
        out_type=[
            jax.ShapeDtypeStruct((chunk_rows, D), jnp.float32),
            jax.ShapeDtypeStruct((chunk_rows, D), jnp.float32),
        ],
        scratch_types=[
            pltpu.VMEM((b_per_w,), jnp.int32),
            pltpu.VMEM((b_per_w,), jnp.int32),
            pltpu.VMEM((b_per_w, D), jnp.float32),
            pltpu.VMEM((b_per_w, D), jnp.float32),
            pltpu.SemaphoreType.DMA,
            pltpu.SemaphoreType.DMA,
        ],
    )
    def gather_k(user_hbm, item_hbm, ut_hbm, it_hbm, ue_out, ie_out,
                 uidx_v, iidx_v, urows_v, irows_v, usem, isem):
        wid = lax.axis_index("s") * NC + lax.axis_index("c")
        src = chunk_off + wid * b_per_w
        dst = wid * b_per_w
        pltpu.sync_copy(user_hbm.at[pl.ds(src, b_per_w)], uidx_v)
        pltpu.sync_copy(item_hbm.at[pl.ds(src, b_per_w)], iidx_v)
        ucp = pltpu.async_copy(ut_hbm.at[uidx_v], urows_v, usem)
        icp = pltpu.async_copy(it_hbm.at[iidx_v], irows_v, isem)
        ucp.wait()
        pltpu.sync_copy(urows_v, ue_out.at[pl.ds(dst, b_per_w)])
        icp.wait()
        pltpu.sync_copy(irows_v, ie_out.at[pl.ds(dst, b_per_w)])

    return gather_k


_sc_gathers = [
    _make_sc_gather(sz, off, NFACT)
    for sz, off in zip(CHUNK_SIZES, (0, CHUNK_SIZES[0]))
]


def _mlp_body(ue, ie, w1a, w1b, b1, w2, b2, w3r, b3, out):
    x = jnp.dot(ue[...], w1a[...], preferred_element_type=jnp.float32)
    x = x + jnp.dot(ie[...], w1b[...], preferred_element_type=jnp.float32)
    h1 = jnp.maximum(x + b1[...], 0.0)
    h2 = jnp.dot(h1, w2[...], preferred_element_type=jnp.float32) + b2[...]
    h2 = jnp.maximum(h2, 0.0)
    o = jax.lax.dot_general(w3r[...], h2, (((1,), (1,)), ((), ())),
                            preferred_element_type=jnp.float32)
    out[...] = o.reshape(out.shape) + b3[0, 0]


def _mlp(ue, ie, W1, b1r, W2, b2r, w3r, b3r):
    B = ue.shape[0]
    block_m = 3072 if B % 3072 == 0 else 2048
    grid = (B // block_m,)
    return pl.pallas_call(
        _mlp_body,
        grid=grid,
        in_specs=[
            pl.BlockSpec((block_m, NFACT), lambda i: (i, 0)),
            pl.BlockSpec((block_m, NFACT), lambda i: (i, 0)),
            pl.BlockSpec((NFACT, H1), lambda i: (0, 0)),
            pl.BlockSpec((NFACT, H1), lambda i: (1, 0)),
            pl.BlockSpec((1, H1), lambda i: (0, 0)),
            pl.BlockSpec((H1, H2), lambda i: (0, 0)),
            pl.BlockSpec((1, H2), lambda i: (0, 0)),
            pl.BlockSpec((1, H2), lambda i: (0, 0)),
            pl.BlockSpec((1, 1), lambda i: (0, 0)),
        ],
        out_specs=pl.BlockSpec((block_m,), lambda i: (i,)),
        out_shape=jax.ShapeDtypeStruct((B,), jnp.float32),
    )(ue, ie, W1, W1, b1r, W2, b2r, w3r, b3r)


@jax.jit
def kernel(user, item, user_table, item_table, W1, b1, W2, b2, W3, b3):
    b1r = b1.reshape(1, H1)
    b2r = b2.reshape(1, H2)
    w3r = W3.reshape(1, H2)
    b3r = b3.reshape(1, 1)
    embs = [g(user, item, user_table, item_table) for g in _sc_gathers]
    outs = [_mlp(ue, ie, W1, b1r, W2, b2r, w3r, b3r) for ue, ie in embs]
    return jnp.concatenate(outs)


# speedup vs baseline: 1.0397x; 1.0277x over previous
import functools

import jax
import jax.numpy as jnp
from jax import lax
from jax.experimental import pallas as pl
from jax.experimental.pallas import tpu as pltpu
from jax.experimental.pallas import tpu_sc as plsc

BATCH = 16384
NFACT = 128
H1 = 512
H2 = 256
CHUNK_SIZES = (8192, 8192)


def _make_sc_gather(chunk_rows, chunk_off, D):
    info = plsc.get_sparse_core_info()
    NC, NS = info.num_cores, info.num_subcores
    NW = NC * NS
    assert chunk_rows % (8 * NW) == 0
    b_per_w = chunk_rows // NW
    mesh = plsc.VectorSubcoreMesh(core_axis_name="c", subcore_axis_name="s")

    @functools.partial(
        pl.kernel,
        mesh=mesh,
        out_type=[
            jax.ShapeDtypeStruct((chunk_rows, D), jnp.float32),
            jax.ShapeDtypeStruct((chunk_rows, D), jnp.float32),
        ],
        scratch_types=[
            pltpu.VMEM((b_per_w,), jnp.int32),
            pltpu.VMEM((b_per_w,), jnp.int32),
            pltpu.VMEM((b_per_w, D), jnp.float32),
            pltpu.VMEM((b_per_w, D), jnp.float32),
            pltpu.SemaphoreType.DMA,
            pltpu.SemaphoreType.DMA,
        ],
    )
    def gather_k(user_hbm, item_hbm, ut_hbm, it_hbm, ue_out, ie_out,
                 uidx_v, iidx_v, urows_v, irows_v, usem, isem):
        wid = lax.axis_index("s") * NC + lax.axis_index("c")
        src = chunk_off + wid * b_per_w
        dst = wid * b_per_w
        pltpu.sync_copy(user_hbm.at[pl.ds(src, b_per_w)], uidx_v)
        pltpu.sync_copy(item_hbm.at[pl.ds(src, b_per_w)], iidx_v)
        ucp = pltpu.async_copy(ut_hbm.at[uidx_v], urows_v, usem)
        icp = pltpu.async_copy(it_hbm.at[iidx_v], irows_v, isem)
        ucp.wait()
        pltpu.sync_copy(urows_v, ue_out.at[pl.ds(dst, b_per_w)])
        icp.wait()
        pltpu.sync_copy(irows_v, ie_out.at[pl.ds(dst, b_per_w)])

    return gather_k


_sc_gathers = [
    _make_sc_gather(sz, off, NFACT)
    for sz, off in zip(CHUNK_SIZES, (0, CHUNK_SIZES[0]))
]


def _mlp_body(ue, ie, w1a, w1b, b1, w2, b2, w3r, b3, out):
    x = jnp.dot(ue[...], w1a[...], preferred_element_type=jnp.float32)
    x = x + jnp.dot(ie[...], w1b[...], preferred_element_type=jnp.float32)
    h1 = jnp.maximum(x + b1[...], 0.0)
    h2 = jnp.dot(h1, w2[...], preferred_element_type=jnp.float32) + b2[...]
    h2 = jnp.maximum(h2, 0.0)
    o = jax.lax.dot_general(w3r[...], h2, (((1,), (1,)), ((), ())),
                            preferred_element_type=jnp.float32)
    out[...] = o.reshape(out.shape) + b3[0, 0]


def _mlp(ue, ie, W1, b1r, W2, b2r, w3r, b3r):
    B = ue.shape[0]
    block_m = 2048
    grid = (B // block_m,)
    return pl.pallas_call(
        _mlp_body,
        grid=grid,
        in_specs=[
            pl.BlockSpec((block_m, NFACT), lambda i: (i, 0)),
            pl.BlockSpec((block_m, NFACT), lambda i: (i, 0)),
            pl.BlockSpec((NFACT, H1), lambda i: (0, 0)),
            pl.BlockSpec((NFACT, H1), lambda i: (1, 0)),
            pl.BlockSpec((1, H1), lambda i: (0, 0)),
            pl.BlockSpec((H1, H2), lambda i: (0, 0)),
            pl.BlockSpec((1, H2), lambda i: (0, 0)),
            pl.BlockSpec((1, H2), lambda i: (0, 0)),
            pl.BlockSpec((1, 1), lambda i: (0, 0)),
        ],
        out_specs=pl.BlockSpec((block_m,), lambda i: (i,)),
        out_shape=jax.ShapeDtypeStruct((B,), jnp.float32),
    )(ue, ie, W1, W1, b1r, W2, b2r, w3r, b3r)


@jax.jit
def kernel(user, item, user_table, item_table, W1, b1, W2, b2, W3, b3):
    b1r = b1.reshape(1, H1)
    b2r = b2.reshape(1, H2)
    w3r = W3.reshape(1, H2)
    b3r = b3.reshape(1, 1)
    embs = [g(user, item, user_table, item_table) for g in _sc_gathers]
    outs = [_mlp(ue, ie, W1, b1r, W2, b2r, w3r, b3r) for ue, ie in embs]
    return jnp.concatenate(outs)


# speedup vs baseline: 1.0485x; 1.0084x over previous
import functools

import jax
import jax.numpy as jnp
from jax import lax
from jax.experimental import pallas as pl
from jax.experimental.pallas import tpu as pltpu
from jax.experimental.pallas import tpu_sc as plsc

BATCH = 16384
NFACT = 128
H1 = 512
H2 = 256
CHUNK_SIZES = (8192, 8192)


def _make_sc_gather(chunk_rows, chunk_off, D):
    info = plsc.get_sparse_core_info()
    NC, NS = info.num_cores, info.num_subcores
    NW = NC * NS
    assert chunk_rows % (8 * NW) == 0
    b_per_w = chunk_rows // NW
    mesh = plsc.VectorSubcoreMesh(core_axis_name="c", subcore_axis_name="s")

    @functools.partial(
        pl.kernel,
        mesh=mesh,
        out_type=[
            jax.ShapeDtypeStruct((chunk_rows, D), jnp.float32),
            jax.ShapeDtypeStruct((chunk_rows, D), jnp.float32),
        ],
        scratch_types=[
            pltpu.VMEM((b_per_w,), jnp.int32),
            pltpu.VMEM((b_per_w,), jnp.int32),
            pltpu.VMEM((b_per_w, D), jnp.float32),
            pltpu.VMEM((b_per_w, D), jnp.float32),
            pltpu.SemaphoreType.DMA,
            pltpu.SemaphoreType.DMA,
        ],
    )
    def gather_k(user_hbm, item_hbm, ut_hbm, it_hbm, ue_out, ie_out,
                 uidx_v, iidx_v, urows_v, irows_v, usem, isem):
        wid = lax.axis_index("s") * NC + lax.axis_index("c")
        src = chunk_off + wid * b_per_w
        dst = wid * b_per_w
        pltpu.sync_copy(user_hbm.at[pl.ds(src, b_per_w)], uidx_v)
        pltpu.sync_copy(item_hbm.at[pl.ds(src, b_per_w)], iidx_v)
        ucp = pltpu.async_copy(ut_hbm.at[uidx_v], urows_v, usem)
        icp = pltpu.async_copy(it_hbm.at[iidx_v], irows_v, isem)
        ucp.wait()
        pltpu.sync_copy(urows_v, ue_out.at[pl.ds(dst, b_per_w)])
        icp.wait()
        pltpu.sync_copy(irows_v, ie_out.at[pl.ds(dst, b_per_w)])

    return gather_k


_sc_gathers = [
    _make_sc_gather(sz, off, NFACT)
    for sz, off in zip(CHUNK_SIZES, (0, CHUNK_SIZES[0]))
]


def _mlp_body(ue, ie, w1a, w1b, b1, w2, b2, w3r, b3, out):
    x = jnp.dot(ue[...], w1a[...], preferred_element_type=jnp.float32)
    x = x + jnp.dot(ie[...], w1b[...], preferred_element_type=jnp.float32)
    h1 = jnp.maximum(x + b1[...], 0.0)
    h2 = jnp.dot(h1, w2[...], preferred_element_type=jnp.float32) + b2[...]
    h2 = jnp.maximum(h2, 0.0)
    o = jax.lax.dot_general(w3r[...], h2, (((1,), (1,)), ((), ())),
                            preferred_element_type=jnp.float32)
    out[...] = o.reshape(out.shape) + b3[0, 0]


def _mlp(ue, ie, W1, b1r, W2, b2r, w3r, b3r):
    B = ue.shape[0]
    block_m = 4096
    grid = (B // block_m,)
    return pl.pallas_call(
        _mlp_body,
        grid=grid,
        in_specs=[
            pl.BlockSpec((block_m, NFACT), lambda i: (i, 0)),
            pl.BlockSpec((block_m, NFACT), lambda i: (i, 0)),
            pl.BlockSpec((NFACT, H1), lambda i: (0, 0)),
            pl.BlockSpec((NFACT, H1), lambda i: (1, 0)),
            pl.BlockSpec((1, H1), lambda i: (0, 0)),
            pl.BlockSpec((H1, H2), lambda i: (0, 0)),
            pl.BlockSpec((1, H2), lambda i: (0, 0)),
            pl.BlockSpec((1, H2), lambda i: (0, 0)),
            pl.BlockSpec((1, 1), lambda i: (0, 0)),
        ],
        out_specs=pl.BlockSpec((block_m,), lambda i: (i,)),
        out_shape=jax.ShapeDtypeStruct((B,), jnp.float32),
    )(ue, ie, W1, W1, b1r, W2, b2r, w3r, b3r)


@jax.jit
def kernel(user, item, user_table, item_table, W1, b1, W2, b2, W3, b3):
    b1r = b1.reshape(1, H1)
    b2r = b2.reshape(1, H2)
    w3r = W3.reshape(1, H2)
    b3r = b3.reshape(1, 1)
    embs = [g(user, item, user_table, item_table) for g in _sc_gathers]
    outs = [_mlp(ue, ie, W1, b1r, W2, b2r, w3r, b3r) for ue, ie in embs]
    return jnp.concatenate(outs)
